# baseline (speedup 1.0000x reference)
import functools

import jax
import jax.numpy as jnp
from jax import lax
from jax.experimental import pallas as pl
from jax.experimental.pallas import tpu as pltpu
from jax.experimental.pallas import tpu_sc as plsc

N = 10000
K = 32
DA = 128
DB = 16
E = N * K

_P_ATOMS = (3200, 3200, 3600)
_P_OFF = (0, 3200, 6400)
_NPART = len(_P_ATOMS)

_NC = 2
_NS = 16
_NW = _NC * _NS
_CHUNK = 400

_BA = 200
_RB = _BA * K
_BN3 = 1000


def _sc_gather(nbr_flat, table, n_edges):
  epw = n_edges // _NW
  nchunk = epw // _CHUNK
  mesh = plsc.VectorSubcoreMesh(core_axis_name="c", subcore_axis_name="s")

  @functools.partial(
      pl.kernel,
      out_type=jax.ShapeDtypeStruct((n_edges, DA), jnp.float32),
      mesh=mesh,
      compiler_params=pltpu.CompilerParams(use_tc_tiling_on_sc=True),
      scratch_types=[
          pltpu.VMEM((epw,), jnp.int32),
          pltpu.VMEM((2, _CHUNK, DA), jnp.float32),
          pltpu.SemaphoreType.DMA,
          pltpu.SemaphoreType.DMA,
          pltpu.SemaphoreType.DMA,
          pltpu.SemaphoreType.DMA,
      ],
  )
  def gather_kernel(idx_hbm, table_hbm, out_hbm, idx_v, rows_v, gs0, gs1, ws0, ws1):
    wid = lax.axis_index("s") * _NC + lax.axis_index("c")
    base = wid * epw
    pltpu.sync_copy(idx_hbm.at[pl.ds(base, epw)], idx_v)
    gsems = (gs0, gs1)
    wsems = (ws0, ws1)

    def issue_gather(j):
      bsel = j % 2
      return pltpu.async_copy(
          table_hbm.at[idx_v.at[pl.ds(j * _CHUNK, _CHUNK)]],
          rows_v.at[bsel], gsems[bsel])

    gathers = [issue_gather(0), None]
    writes = [None, None]
    for j in range(nchunk):
      bsel = j % 2
      nb = (j + 1) % 2
      if j + 1 < nchunk:
        if writes[nb] is not None:
          writes[nb].wait()
        gathers[nb] = issue_gather(j + 1)
      gathers[bsel].wait()
      writes[bsel] = pltpu.async_copy(
          rows_v.at[bsel], out_hbm.at[pl.ds(base + j * _CHUNK, _CHUNK)],
          wsems[bsel])
    writes[0].wait()
    writes[1].wait()

  return gather_kernel(nbr_flat, table)


def _edge_t(g_ref, bond_ref, atom_ref, ws_ref, wn_ref, wb_ref, b_ref):
  s = jnp.dot(atom_ref[...], ws_ref[...], preferred_element_type=jnp.float32)
  s = s + b_ref[...]
  t = jnp.dot(g_ref[...], wn_ref[...], preferred_element_type=jnp.float32)
  t = t + jnp.dot(bond_ref[...], wb_ref[...], preferred_element_type=jnp.float32)
  return t.reshape(_BA, K, 2 * DA) + s[:, None, :]


def _pass1_body(g_ref, bond_ref, atom_ref, ws_ref, wn_ref, wb_ref, b_ref,
                sum_ref, sumsq_ref):
  t3 = _edge_t(g_ref, bond_ref, atom_ref, ws_ref, wn_ref, wb_ref, b_ref)

  @pl.when(pl.program_id(0) == 0)
  def _init():
    sum_ref[...] = jnp.zeros_like(sum_ref)
    sumsq_ref[...] = jnp.zeros_like(sumsq_ref)

  sum_ref[...] += jnp.sum(t3, axis=(0, 1))[None, :]
  sumsq_ref[...] += jnp.sum(t3 * t3, axis=(0, 1))[None, :]


def _pass2_body(g_ref, bond_ref, atom_ref, ws_ref, wn_ref, wb_ref, b_ref,
                ns_ref, sum2_ref, sumsq2_ref):
  u = _edge_t(g_ref, bond_ref, atom_ref, ws_ref, wn_ref, wb_ref, b_ref)
  filt = 1.0 / (1.0 + jnp.exp(-u[:, :, :DA]))
  core = jnp.log(1.0 + jnp.exp(u[:, :, DA:]))
  ns = jnp.sum(filt * core, axis=1)
  ns_ref[...] = ns

  @pl.when(pl.program_id(0) == 0)
  def _init():
    sum2_ref[...] = jnp.zeros_like(sum2_ref)
    sumsq2_ref[...] = jnp.zeros_like(sumsq2_ref)

  sum2_ref[...] += jnp.sum(ns, axis=0)[None, :]
  sumsq2_ref[...] += jnp.sum(ns * ns, axis=0)[None, :]


def _pass3_body(atom_ref, ns_ref, a2_ref, c2_ref, out_ref):
  out_ref[...] = jax.nn.softplus(
      atom_ref[...] + ns_ref[...] * a2_ref[...] + c2_ref[...])


_full = lambda shp: pl.BlockSpec(shp, lambda i: (0, 0))

_EDGE_IN = [
    pl.BlockSpec((_RB, DA), lambda i: (i, 0)),
    pl.BlockSpec((_RB, DB), lambda i: (i, 0)),
    pl.BlockSpec((_BA, DA), lambda i: (i, 0)),
    _full((DA, 2 * DA)),
    _full((DA, 2 * DA)),
    _full((DB, 2 * DA)),
    _full((1, 2 * DA)),
]


def kernel(neighbor_indices, atom_features, bond_features, W, b,
           bn1_scale, bn1_offset, bn2_scale, bn2_offset):
  nbr_flat = neighbor_indices.astype(jnp.int32).reshape(E)
  bond_flat = bond_features.reshape(E, DB)
  w_self = W[:DA]
  w_nbr = W[DA:2 * DA]
  w_bond = W[2 * DA:]
  b2 = b.reshape(1, 2 * DA)

  nbr_h = [lax.slice_in_dim(nbr_flat, _P_OFF[p] * K, (_P_OFF[p] + _P_ATOMS[p]) * K)
           for p in range(_NPART)]
  bond_h = [lax.slice_in_dim(bond_flat, _P_OFF[p] * K, (_P_OFF[p] + _P_ATOMS[p]) * K)
            for p in range(_NPART)]
  atom_h = [lax.slice_in_dim(atom_features, _P_OFF[p], _P_OFF[p] + _P_ATOMS[p])
            for p in range(_NPART)]

  g_h = [_sc_gather(nbr_h[p], atom_features, _P_ATOMS[p] * K)
         for p in range(_NPART)]

  def pass1(p):
    return pl.pallas_call(
        _pass1_body,
        grid=(_P_ATOMS[p] // _BA,),
        in_specs=_EDGE_IN,
        out_specs=[_full((1, 2 * DA)), _full((1, 2 * DA))],
        out_shape=[jax.ShapeDtypeStruct((1, 2 * DA), jnp.float32)] * 2,
    )(g_h[p], bond_h[p], atom_h[p], w_self, w_nbr, w_bond, b2)

  sums = [pass1(p) for p in range(_NPART)]
  colsum = sum(s[0] for s in sums)
  colsumsq = sum(s[1] for s in sums)

  mean1 = colsum / E
  var1 = colsumsq / E - mean1 * mean1
  a1 = bn1_scale.reshape(1, 2 * DA) * lax.rsqrt(var1 + 1e-5)
  c1 = bn1_offset.reshape(1, 2 * DA) - mean1 * a1

  ws_2 = w_self * a1
  wn_2 = w_nbr * a1
  wb_2 = w_bond * a1
  bias_2 = b2 * a1 + c1

  def pass2(p):
    return pl.pallas_call(
        _pass2_body,
        grid=(_P_ATOMS[p] // _BA,),
        in_specs=_EDGE_IN,
        out_specs=[
            pl.BlockSpec((_BA, DA), lambda i: (i, 0)),
            _full((1, DA)),
            _full((1, DA)),
        ],
        out_shape=[
            jax.ShapeDtypeStruct((_P_ATOMS[p], DA), jnp.float32),
            jax.ShapeDtypeStruct((1, DA), jnp.float32),
            jax.ShapeDtypeStruct((1, DA), jnp.float32),
        ],
    )(g_h[p], bond_h[p], atom_h[p], ws_2, wn_2, wb_2, bias_2)

  outs2 = [pass2(p) for p in range(_NPART)]
  ns = jnp.concatenate([o[0] for o in outs2], axis=0)
  colsum2 = sum(o[1] for o in outs2)
  colsumsq2 = sum(o[2] for o in outs2)

  mean2 = colsum2 / N
  var2 = colsumsq2 / N - mean2 * mean2
  a2 = bn2_scale.reshape(1, DA) * lax.rsqrt(var2 + 1e-5)
  c2 = bn2_offset.reshape(1, DA) - mean2 * a2

  out = pl.pallas_call(
      _pass3_body,
      grid=(N // _BN3,),
      in_specs=[
          pl.BlockSpec((_BN3, DA), lambda i: (i, 0)),
          pl.BlockSpec((_BN3, DA), lambda i: (i, 0)),
          _full((1, DA)),
          _full((1, DA)),
      ],
      out_specs=pl.BlockSpec((_BN3, DA), lambda i: (i, 0)),
      out_shape=jax.ShapeDtypeStruct((N, DA), jnp.float32),
  )(atom_features, ns, a2, c2)

  return out


# speedup vs baseline: 1.1528x; 1.1528x over previous
import functools

import jax
import jax.numpy as jnp
from jax import lax
from jax.experimental import pallas as pl
from jax.experimental.pallas import tpu as pltpu
from jax.experimental.pallas import tpu_sc as plsc

N = 10000
K = 32
DA = 128
DB = 16
E = N * K

_P_ATOMS = (5000, 5000)
_P_OFF = (0, 5000)
_NPART = len(_P_ATOMS)

_NC = 2
_NS = 16
_NW = _NC * _NS
_CHUNK = 200

_BA = 200
_RB = _BA * K
_BN3 = 1000


def _sc_gather(nbr_flat, table, n_edges):
  epw = n_edges // _NW
  nchunk = epw // _CHUNK
  mesh = plsc.VectorSubcoreMesh(core_axis_name="c", subcore_axis_name="s")

  @functools.partial(
      pl.kernel,
      out_type=jax.ShapeDtypeStruct((n_edges, DA), jnp.float32),
      mesh=mesh,
      compiler_params=pltpu.CompilerParams(use_tc_tiling_on_sc=True),
      scratch_types=[
          pltpu.VMEM((epw,), jnp.int32),
          pltpu.VMEM((2, _CHUNK, DA), jnp.float32),
          pltpu.SemaphoreType.DMA,
          pltpu.SemaphoreType.DMA,
          pltpu.SemaphoreType.DMA,
          pltpu.SemaphoreType.DMA,
      ],
  )
  def gather_kernel(idx_hbm, table_hbm, out_hbm, idx_v, rows_v, gs0, gs1, ws0, ws1):
    wid = lax.axis_index("s") * _NC + lax.axis_index("c")
    base = wid * epw
    pltpu.sync_copy(idx_hbm.at[pl.ds(base, epw)], idx_v)
    gsems = (gs0, gs1)
    wsems = (ws0, ws1)

    def issue_gather(j):
      bsel = j % 2
      return pltpu.async_copy(
          table_hbm.at[idx_v.at[pl.ds(j * _CHUNK, _CHUNK)]],
          rows_v.at[bsel], gsems[bsel])

    gathers = [issue_gather(0), None]
    writes = [None, None]
    for j in range(nchunk):
      bsel = j % 2
      nb = (j + 1) % 2
      if j + 1 < nchunk:
        if writes[nb] is not None:
          writes[nb].wait()
        gathers[nb] = issue_gather(j + 1)
      gathers[bsel].wait()
      writes[bsel] = pltpu.async_copy(
          rows_v.at[bsel], out_hbm.at[pl.ds(base + j * _CHUNK, _CHUNK)],
          wsems[bsel])
    writes[0].wait()
    writes[1].wait()

  return gather_kernel(nbr_flat, table)


def _edge_t(g_ref, bond_ref, atom_ref, ws_ref, wn_ref, wb_ref, b_ref):
  s = jnp.dot(atom_ref[...], ws_ref[...], preferred_element_type=jnp.float32)
  s = s + b_ref[...]
  t = jnp.dot(g_ref[...], wn_ref[...], preferred_element_type=jnp.float32)
  t = t + jnp.dot(bond_ref[...], wb_ref[...], preferred_element_type=jnp.float32)
  return t.reshape(_BA, K, 2 * DA) + s[:, None, :]


def _pass1_body(g_ref, bond_ref, atom_ref, ws_ref, wn_ref, wb_ref, b_ref,
                sum_ref, sumsq_ref):
  t3 = _edge_t(g_ref, bond_ref, atom_ref, ws_ref, wn_ref, wb_ref, b_ref)

  @pl.when(pl.program_id(0) == 0)
  def _init():
    sum_ref[...] = jnp.zeros_like(sum_ref)
    sumsq_ref[...] = jnp.zeros_like(sumsq_ref)

  sum_ref[...] += jnp.sum(t3, axis=(0, 1))[None, :]
  sumsq_ref[...] += jnp.sum(t3 * t3, axis=(0, 1))[None, :]


def _pass2_body(g_ref, bond_ref, atom_ref, ws_ref, wn_ref, wb_ref, b_ref,
                ns_ref, sum2_ref, sumsq2_ref):
  u = _edge_t(g_ref, bond_ref, atom_ref, ws_ref, wn_ref, wb_ref, b_ref)
  filt = 1.0 / (1.0 + jnp.exp(-u[:, :, :DA]))
  core = jnp.log(1.0 + jnp.exp(u[:, :, DA:]))
  ns = jnp.sum(filt * core, axis=1)
  ns_ref[...] = ns

  @pl.when(pl.program_id(0) == 0)
  def _init():
    sum2_ref[...] = jnp.zeros_like(sum2_ref)
    sumsq2_ref[...] = jnp.zeros_like(sumsq2_ref)

  sum2_ref[...] += jnp.sum(ns, axis=0)[None, :]
  sumsq2_ref[...] += jnp.sum(ns * ns, axis=0)[None, :]


def _pass3_body(atom_ref, ns_ref, a2_ref, c2_ref, out_ref):
  out_ref[...] = jax.nn.softplus(
      atom_ref[...] + ns_ref[...] * a2_ref[...] + c2_ref[...])


_full = lambda shp: pl.BlockSpec(shp, lambda i: (0, 0))

def _edge_in(bo):
  return [
      pl.BlockSpec((_RB, DA), lambda i: (i, 0)),
      pl.BlockSpec((_RB, DB), lambda i, bo=bo: (i + bo, 0)),
      pl.BlockSpec((_BA, DA), lambda i, bo=bo: (i + bo, 0)),
      _full((DA, 2 * DA)),
      _full((DA, 2 * DA)),
      _full((DB, 2 * DA)),
      _full((1, 2 * DA)),
  ]


def kernel(neighbor_indices, atom_features, bond_features, W, b,
           bn1_scale, bn1_offset, bn2_scale, bn2_offset):
  nbr_flat = neighbor_indices.astype(jnp.int32).reshape(E)
  bond_flat = bond_features.reshape(E, DB)
  w_self = W[:DA]
  w_nbr = W[DA:2 * DA]
  w_bond = W[2 * DA:]
  b2 = b.reshape(1, 2 * DA)

  nbr_h = [lax.slice_in_dim(nbr_flat, _P_OFF[p] * K, (_P_OFF[p] + _P_ATOMS[p]) * K)
           for p in range(_NPART)]

  g_h = [_sc_gather(nbr_h[p], atom_features, _P_ATOMS[p] * K)
         for p in range(_NPART)]

  def pass1(p):
    return pl.pallas_call(
        _pass1_body,
        grid=(_P_ATOMS[p] // _BA,),
        in_specs=_edge_in(_P_OFF[p] // _BA),
        out_specs=[_full((1, 2 * DA)), _full((1, 2 * DA))],
        out_shape=[jax.ShapeDtypeStruct((1, 2 * DA), jnp.float32)] * 2,
    )(g_h[p], bond_flat, atom_features, w_self, w_nbr, w_bond, b2)

  sums = [pass1(p) for p in range(_NPART)]
  colsum = sum(s[0] for s in sums)
  colsumsq = sum(s[1] for s in sums)

  mean1 = colsum / E
  var1 = colsumsq / E - mean1 * mean1
  a1 = bn1_scale.reshape(1, 2 * DA) * lax.rsqrt(var1 + 1e-5)
  c1 = bn1_offset.reshape(1, 2 * DA) - mean1 * a1

  ws_2 = w_self * a1
  wn_2 = w_nbr * a1
  wb_2 = w_bond * a1
  bias_2 = b2 * a1 + c1

  def pass2(p):
    return pl.pallas_call(
        _pass2_body,
        grid=(_P_ATOMS[p] // _BA,),
        in_specs=_edge_in(_P_OFF[p] // _BA),
        out_specs=[
            pl.BlockSpec((_BA, DA), lambda i: (i, 0)),
            _full((1, DA)),
            _full((1, DA)),
        ],
        out_shape=[
            jax.ShapeDtypeStruct((_P_ATOMS[p], DA), jnp.float32),
            jax.ShapeDtypeStruct((1, DA), jnp.float32),
            jax.ShapeDtypeStruct((1, DA), jnp.float32),
        ],
    )(g_h[p], bond_flat, atom_features, ws_2, wn_2, wb_2, bias_2)

  outs2 = [pass2(p) for p in range(_NPART)]
  ns = jnp.concatenate([o[0] for o in outs2], axis=0)
  colsum2 = sum(o[1] for o in outs2)
  colsumsq2 = sum(o[2] for o in outs2)

  mean2 = colsum2 / N
  var2 = colsumsq2 / N - mean2 * mean2
  a2 = bn2_scale.reshape(1, DA) * lax.rsqrt(var2 + 1e-5)
  c2 = bn2_offset.reshape(1, DA) - mean2 * a2

  out = pl.pallas_call(
      _pass3_body,
      grid=(N // _BN3,),
      in_specs=[
          pl.BlockSpec((_BN3, DA), lambda i: (i, 0)),
          pl.BlockSpec((_BN3, DA), lambda i: (i, 0)),
          _full((1, DA)),
          _full((1, DA)),
      ],
      out_specs=pl.BlockSpec((_BN3, DA), lambda i: (i, 0)),
      out_shape=jax.ShapeDtypeStruct((N, DA), jnp.float32),
  )(atom_features, ns, a2, c2)

  return out


# speedup vs baseline: 1.1564x; 1.0031x over previous
import functools

import jax
import jax.numpy as jnp
from jax import lax
from jax.experimental import pallas as pl
from jax.experimental.pallas import tpu as pltpu
from jax.experimental.pallas import tpu_sc as plsc

N = 10000
K = 32
DA = 128
DB = 16
E = N * K

_P_ATOMS = (5000, 5000)
_P_OFF = (0, 5000)
_NPART = len(_P_ATOMS)

_NC = 2
_NS = 16
_NW = _NC * _NS
_CHUNK = 200

_BA = 200
_RB = _BA * K
_BN3 = 1000


def _sc_gather(nbr2d, row0_global, n_edges, table):
  n_rows = n_edges // 128
  rpw = (n_rows + _NW - 1) // _NW
  rpw += (-rpw) % 8
  last_rows = n_rows - (_NW - 1) * rpw
  off = row0_global % 8
  assert 0 < last_rows <= rpw and rpw % 8 == 0
  mesh = plsc.VectorSubcoreMesh(core_axis_name="c", subcore_axis_name="s")

  @functools.partial(
      pl.kernel,
      out_type=jax.ShapeDtypeStruct((n_edges, DA), jnp.float32),
      mesh=mesh,
      compiler_params=pltpu.CompilerParams(use_tc_tiling_on_sc=True),
      scratch_types=[
          pltpu.VMEM((rpw + 8, 128), jnp.int32),
          pltpu.VMEM((2, 128, DA), jnp.float32),
          pltpu.SemaphoreType.DMA,
          pltpu.SemaphoreType.DMA,
          pltpu.SemaphoreType.DMA,
          pltpu.SemaphoreType.DMA,
      ],
  )
  def gather_kernel(idx_hbm, table_hbm, out_hbm, idx_v, rows_v, gs0, gs1, ws0, ws1):
    wid = lax.axis_index("s") * _NC + lax.axis_index("c")
    row0 = row0_global - off + wid * rpw
    gsems = (gs0, gs1)
    wsems = (ws0, ws1)

    def run(nrows):
      stage = nrows + off + (-(nrows + off)) % 8
      pltpu.sync_copy(idx_hbm.at[pl.ds(row0, stage)],
                      idx_v.at[pl.ds(0, stage)])

      def issue_gather(j):
        bsel = j % 2
        return pltpu.async_copy(
            table_hbm.at[idx_v.at[j + off]], rows_v.at[bsel], gsems[bsel])

      gathers = [issue_gather(0), None]
      writes = [None, None]
      for j in range(nrows):
        bsel = j % 2
        nb = (j + 1) % 2
        if j + 1 < nrows:
          if writes[nb] is not None:
            writes[nb].wait()
          gathers[nb] = issue_gather(j + 1)
        gathers[bsel].wait()
        writes[bsel] = pltpu.async_copy(
            rows_v.at[bsel],
            out_hbm.at[pl.ds((wid * rpw + j) * 128, 128)],
            wsems[bsel])
      writes[nrows % 2].wait()
      if nrows > 1:
        writes[(nrows - 1) % 2].wait()

    @pl.when(wid < _NW - 1)
    def _main():
      run(rpw)

    @pl.when(wid == _NW - 1)
    def _tail():
      run(last_rows)

  return gather_kernel(nbr2d, table)


def _edge_t(g_ref, bond_ref, atom_ref, ws_ref, wn_ref, wb_ref, b_ref):
  s = jnp.dot(atom_ref[...], ws_ref[...], preferred_element_type=jnp.float32)
  s = s + b_ref[...]
  t = jnp.dot(g_ref[...], wn_ref[...], preferred_element_type=jnp.float32)
  t = t + jnp.dot(bond_ref[...], wb_ref[...], preferred_element_type=jnp.float32)
  return t.reshape(_BA, K, 2 * DA) + s[:, None, :]


def _pass1_body(g_ref, bond_ref, atom_ref, ws_ref, wn_ref, wb_ref, b_ref,
                sum_ref, sumsq_ref):
  t3 = _edge_t(g_ref, bond_ref, atom_ref, ws_ref, wn_ref, wb_ref, b_ref)

  @pl.when(pl.program_id(0) == 0)
  def _init():
    sum_ref[...] = jnp.zeros_like(sum_ref)
    sumsq_ref[...] = jnp.zeros_like(sumsq_ref)

  sum_ref[...] += jnp.sum(t3, axis=(0, 1))[None, :]
  sumsq_ref[...] += jnp.sum(t3 * t3, axis=(0, 1))[None, :]


def _pass2_body(g_ref, bond_ref, atom_ref, ws_ref, wn_ref, wb_ref, b_ref,
                ns_ref, sum2_ref, sumsq2_ref):
  u = _edge_t(g_ref, bond_ref, atom_ref, ws_ref, wn_ref, wb_ref, b_ref)
  filt = 0.5 * jnp.tanh(0.5 * u[:, :, :DA]) + 0.5
  core = jnp.log(1.0 + jnp.exp(u[:, :, DA:]))
  ns = jnp.sum(filt * core, axis=1)
  ns_ref[...] = ns

  @pl.when(pl.program_id(0) == 0)
  def _init():
    sum2_ref[...] = jnp.zeros_like(sum2_ref)
    sumsq2_ref[...] = jnp.zeros_like(sumsq2_ref)

  sum2_ref[...] += jnp.sum(ns, axis=0)[None, :]
  sumsq2_ref[...] += jnp.sum(ns * ns, axis=0)[None, :]


def _pass3_body(atom_ref, ns_ref, a2_ref, c2_ref, out_ref):
  out_ref[...] = jax.nn.softplus(
      atom_ref[...] + ns_ref[...] * a2_ref[...] + c2_ref[...])


_full = lambda shp: pl.BlockSpec(shp, lambda i: (0, 0))

def _edge_in(bo):
  return [
      pl.BlockSpec((_RB, DA), lambda i: (i, 0)),
      pl.BlockSpec((_RB, DB), lambda i, bo=bo: (i + bo, 0)),
      pl.BlockSpec((_BA, DA), lambda i, bo=bo: (i + bo, 0)),
      _full((DA, 2 * DA)),
      _full((DA, 2 * DA)),
      _full((DB, 2 * DA)),
      _full((1, 2 * DA)),
  ]


def kernel(neighbor_indices, atom_features, bond_features, W, b,
           bn1_scale, bn1_offset, bn2_scale, bn2_offset):
  nbr_flat = neighbor_indices.astype(jnp.int32).reshape(E)
  bond_flat = bond_features.reshape(E, DB)
  w_self = W[:DA]
  w_nbr = W[DA:2 * DA]
  w_bond = W[2 * DA:]
  b2 = b.reshape(1, 2 * DA)

  nbr2d = jnp.concatenate(
      [nbr_flat, jnp.zeros((512,), jnp.int32)]).reshape(E // 128 + 4, 128)

  g_h = [_sc_gather(nbr2d, _P_OFF[p] * K // 128, _P_ATOMS[p] * K, atom_features)
         for p in range(_NPART)]

  def pass1(p):
    return pl.pallas_call(
        _pass1_body,
        grid=(_P_ATOMS[p] // _BA,),
        in_specs=_edge_in(_P_OFF[p] // _BA),
        out_specs=[_full((1, 2 * DA)), _full((1, 2 * DA))],
        out_shape=[jax.ShapeDtypeStruct((1, 2 * DA), jnp.float32)] * 2,
    )(g_h[p], bond_flat, atom_features, w_self, w_nbr, w_bond, b2)

  sums = [pass1(p) for p in range(_NPART)]
  colsum = sum(s[0] for s in sums)
  colsumsq = sum(s[1] for s in sums)

  mean1 = colsum / E
  var1 = colsumsq / E - mean1 * mean1
  a1 = bn1_scale.reshape(1, 2 * DA) * lax.rsqrt(var1 + 1e-5)
  c1 = bn1_offset.reshape(1, 2 * DA) - mean1 * a1

  ws_2 = w_self * a1
  wn_2 = w_nbr * a1
  wb_2 = w_bond * a1
  bias_2 = b2 * a1 + c1

  def pass2(p):
    return pl.pallas_call(
        _pass2_body,
        grid=(_P_ATOMS[p] // _BA,),
        in_specs=_edge_in(_P_OFF[p] // _BA),
        out_specs=[
            pl.BlockSpec((_BA, DA), lambda i: (i, 0)),
            _full((1, DA)),
            _full((1, DA)),
        ],
        out_shape=[
            jax.ShapeDtypeStruct((_P_ATOMS[p], DA), jnp.float32),
            jax.ShapeDtypeStruct((1, DA), jnp.float32),
            jax.ShapeDtypeStruct((1, DA), jnp.float32),
        ],
    )(g_h[p], bond_flat, atom_features, ws_2, wn_2, wb_2, bias_2)

  outs2 = [pass2(p) for p in range(_NPART)]
  ns = jnp.concatenate([o[0] for o in outs2], axis=0)
  colsum2 = sum(o[1] for o in outs2)
  colsumsq2 = sum(o[2] for o in outs2)

  mean2 = colsum2 / N
  var2 = colsumsq2 / N - mean2 * mean2
  a2 = bn2_scale.reshape(1, DA) * lax.rsqrt(var2 + 1e-5)
  c2 = bn2_offset.reshape(1, DA) - mean2 * a2

  out = pl.pallas_call(
      _pass3_body,
      grid=(N // _BN3,),
      in_specs=[
          pl.BlockSpec((_BN3, DA), lambda i: (i, 0)),
          pl.BlockSpec((_BN3, DA), lambda i: (i, 0)),
          _full((1, DA)),
          _full((1, DA)),
      ],
      out_specs=pl.BlockSpec((_BN3, DA), lambda i: (i, 0)),
      out_shape=jax.ShapeDtypeStruct((N, DA), jnp.float32),
  )(atom_features, ns, a2, c2)

  return out
